# baseline (speedup 1.0000x reference)
import functools

import jax
import jax.numpy as jnp
from jax import lax
from jax.experimental import pallas as pl
from jax.experimental.pallas import tpu as pltpu
from jax.experimental.pallas import tpu_sc as plsc

N = 10000
E = 160000
D = 256
G = 64
HD = D // 2
NC, NS = 2, 16
CH = 64
NCHUNK = 2560
EPAD = NCHUNK * CH
RPT = NCHUNK // NS
NPAD = 10112
ZPT = NPAD // NS


NBUF = 4
PHASES = 4
PRPT = RPT // PHASES
NG = PRPT // NBUF


def _sc_agg_body(x2_hbm, src3_hbm, dst3_hbm, zeros_hbm, out_hbm,
                 idx_s, idx_d, rows, acc, *sems):
  gsem = sems[:NBUF]
  ssem = sems[NBUF:]
  c = lax.axis_index("c")
  s = lax.axis_index("s")
  pltpu.sync_copy(zeros_hbm.at[pl.ds(s * ZPT, ZPT)], acc.at[pl.ds(s * ZPT, ZPT)])
  base = s * RPT
  plsc.subcore_barrier()

  for ph in range(PHASES):
    hbase = base + ph * PRPT
    pltpu.sync_copy(src3_hbm.at[c, pl.ds(hbase, PRPT)], idx_s)
    pltpu.sync_copy(dst3_hbm.at[pl.ds(hbase, PRPT)], idx_d)
    pltpu.async_copy(x2_hbm.at[idx_s.at[0]], rows.at[0], gsem[0])
    pltpu.async_copy(x2_hbm.at[idx_s.at[1]], rows.at[1], gsem[1])
    pltpu.async_copy(x2_hbm.at[idx_s.at[2]], rows.at[2], gsem[2])

    def grp(g, carry):
      for b in range(NBUF):
        j = g * NBUF + b
        b3 = (b + 3) % NBUF
        pltpu.make_async_copy(x2_hbm.at[idx_s.at[j]], rows.at[b],
                              gsem[b]).wait()
        pltpu.async_copy(rows.at[b], acc.at[idx_d.at[j]], ssem[b], add=True)

        @pl.when(j >= 1)
        def _():
          pltpu.make_async_copy(rows.at[b3], acc.at[idx_d.at[j - 1]],
                                ssem[b3]).wait()

        @pl.when(j + 3 < PRPT)
        def _():
          pltpu.async_copy(x2_hbm.at[idx_s.at[j + 3]], rows.at[b3], gsem[b3])
      return carry

    lax.fori_loop(0, NG, grp, 0)
    for j in range(PRPT - 1, PRPT):
      pltpu.make_async_copy(rows.at[j % NBUF], acc.at[idx_d.at[j]],
                            ssem[j % NBUF]).wait()

  plsc.subcore_barrier()
  pltpu.sync_copy(acc.at[pl.ds(s * ZPT, ZPT)], out_hbm.at[c, pl.ds(s * ZPT, ZPT)])


_sc_agg = pl.kernel(
    _sc_agg_body,
    out_type=jax.ShapeDtypeStruct((NC, NPAD, HD), jnp.float32),
    mesh=plsc.VectorSubcoreMesh(core_axis_name="c", subcore_axis_name="s",
                                num_cores=NC, num_subcores=NS),
    scratch_types=[
        pltpu.VMEM((PRPT, CH), jnp.int32),
        pltpu.VMEM((PRPT, CH), jnp.int32),
        pltpu.VMEM((NBUF, CH, HD), jnp.float32),
        pltpu.VMEM_SHARED((NPAD, HD), jnp.float32),
    ] + [pltpu.SemaphoreType.DMA] * (2 * NBUF),
)

BN = 1000


def _mlp_body(xa_ref, xb_ref, aa_ref, ab_ref, w1_ref, b1_ref, w2_ref, b2_ref,
              oa_ref, ob_ref, *, act):
  h = jnp.concatenate([xa_ref[...] + aa_ref[...], xb_ref[...] + ab_ref[...]],
                      axis=1)
  h1 = jnp.dot(h, w1_ref[...], preferred_element_type=jnp.float32) + b1_ref[...]
  h1 = jnp.maximum(h1, 0.0)
  o = jnp.dot(h1, w2_ref[...], preferred_element_type=jnp.float32) + b2_ref[...]
  if act:
    o = jnp.maximum(o, 0.0)
  oa_ref[...] = o[:, :HD]
  ob_ref[...] = o[:, HD:]


def _make_mlp(act):
  row = pl.BlockSpec((BN, HD), lambda i: (i, 0))
  def full(shape):
    return pl.BlockSpec(shape, lambda i: (0, 0))
  return pl.pallas_call(
      functools.partial(_mlp_body, act=act),
      grid=(N // BN,),
      in_specs=[row, row, row, row,
                full((D, D)), full((1, D)), full((D, D)), full((1, D))],
      out_specs=[row, row],
      out_shape=[jax.ShapeDtypeStruct((N, HD), jnp.float32)] * 2,
  )


_mlp_act = _make_mlp(True)
_mlp_lin = _make_mlp(False)


def _pool_body(b_ref, ha_ref, hb_ref, out_ref, acc_ref):
  i = pl.program_id(0)

  @pl.when(i == 0)
  def _():
    acc_ref[...] = jnp.zeros_like(acc_ref)

  oh = (b_ref[...] == lax.broadcasted_iota(jnp.int32, (BN, G), 1)
        ).astype(jnp.float32)
  hx = jnp.concatenate(
      [ha_ref[...], hb_ref[...], jnp.ones((BN, 128), jnp.float32)], axis=1)
  acc_ref[...] += lax.dot_general(oh, hx, (((0,), (0,)), ((), ())),
                                  preferred_element_type=jnp.float32)

  @pl.when(i == pl.num_programs(0) - 1)
  def _():
    out_ref[...] = acc_ref[:, :D] / jnp.maximum(acc_ref[:, D:D + 1], 1.0)


_pool = pl.pallas_call(
    _pool_body,
    grid=(N // BN,),
    in_specs=[pl.BlockSpec((BN, 1), lambda i: (i, 0)),
              pl.BlockSpec((BN, HD), lambda i: (i, 0)),
              pl.BlockSpec((BN, HD), lambda i: (i, 0))],
    out_specs=pl.BlockSpec((G, D), lambda i: (0, 0)),
    out_shape=jax.ShapeDtypeStruct((G, D), jnp.float32),
    scratch_shapes=[pltpu.VMEM((G, D + 128), jnp.float32)],
)


def kernel(x, edge_index, batch, W1_0, b1_0, W2_0, b2_0,
           W1_1, b1_1, W2_1, b2_1, W1_2, b1_2, W2_2, b2_2):
  src = edge_index[0].astype(jnp.int32)
  dst = edge_index[1].astype(jnp.int32)
  srcp = jnp.concatenate([src, jnp.zeros((EPAD - E,), jnp.int32)])
  dstp = jnp.concatenate([dst, jnp.full((EPAD - E,), N, jnp.int32)])
  src3 = jnp.stack([srcp, srcp + N]).reshape(NC, NCHUNK, CH)
  dst3 = dstp.reshape(NCHUNK, CH)
  zeros = jnp.zeros((NPAD, HD), jnp.float32)

  params = [(W1_0, b1_0, W2_0, b2_0), (W1_1, b1_1, W2_1, b2_1),
            (W1_2, b1_2, W2_2, b2_2)]
  xa, xb = x[:, :HD], x[:, HD:]
  for l, (w1, b1, w2, b2) in enumerate(params):
    x2 = jnp.concatenate([xa, xb], axis=0)
    agg = _sc_agg(x2, src3, dst3, zeros)
    mlp = _mlp_act if l < 2 else _mlp_lin
    xa, xb = mlp(xa, xb, agg[0, :N], agg[1, :N],
                 w1, b1.reshape(1, D), w2, b2.reshape(1, D))
  return _pool(batch[:, None].astype(jnp.int32), xa, xb)


# speedup vs baseline: 1.0342x; 1.0342x over previous
import functools

import jax
import jax.numpy as jnp
from jax import lax
from jax.experimental import pallas as pl
from jax.experimental.pallas import tpu as pltpu
from jax.experimental.pallas import tpu_sc as plsc

N = 10000
E = 160000
D = 256
G = 64
HD = D // 2
NC, NS = 2, 16
CH = 128
NCHUNK = 1280
EPAD = NCHUNK * CH
RPT = NCHUNK // NS
NPAD = 10112
ZPT = NPAD // NS
NBUF = 2
PHASES = 2
PRPT = RPT // PHASES
NG = PRPT // NBUF


def _sc_agg_body(xa_hbm, xb_hbm, src3_hbm, dst3_hbm, zeros_hbm, out_hbm,
                 idx_s, idx_d, rows, acc, *sems):
  c = lax.axis_index("c")
  s = lax.axis_index("s")
  pltpu.sync_copy(zeros_hbm.at[pl.ds(s * ZPT, ZPT)], acc.at[pl.ds(s * ZPT, ZPT)])
  base = s * RPT
  plsc.subcore_barrier()

  def gather(j, b):
    @pl.when(c == 0)
    def _():
      pltpu.async_copy(xa_hbm.at[idx_s.at[j]], rows.at[b], sems[b])

    @pl.when(c == 1)
    def _():
      pltpu.async_copy(xb_hbm.at[idx_s.at[j]], rows.at[b], sems[b])

  for ph in range(PHASES):
    hbase = base + ph * PRPT
    pltpu.sync_copy(src3_hbm.at[pl.ds(hbase, PRPT)], idx_s)
    pltpu.sync_copy(dst3_hbm.at[pl.ds(hbase, PRPT)], idx_d)
    for b in range(NBUF):
      gather(b, b)

    def grp(g, carry):
      for b in range(NBUF):
        j = g * NBUF + b
        pltpu.make_async_copy(xa_hbm.at[idx_s.at[j]], rows.at[b],
                              sems[b]).wait()
        pltpu.sync_copy(rows.at[b], acc.at[idx_d.at[j]], add=True)

        @pl.when(g < NG - 1)
        def _():
          gather(j + NBUF, b)
      return carry

    lax.fori_loop(0, NG, grp, 0)

  plsc.subcore_barrier()
  pltpu.sync_copy(acc.at[pl.ds(s * ZPT, ZPT)], out_hbm.at[c, pl.ds(s * ZPT, ZPT)])


_sc_agg = pl.kernel(
    _sc_agg_body,
    out_type=jax.ShapeDtypeStruct((NC, NPAD, HD), jnp.float32),
    mesh=plsc.VectorSubcoreMesh(core_axis_name="c", subcore_axis_name="s",
                                num_cores=NC, num_subcores=NS),
    scratch_types=[
        pltpu.VMEM((PRPT, CH), jnp.int32),
        pltpu.VMEM((PRPT, CH), jnp.int32),
        pltpu.VMEM((NBUF, CH, HD), jnp.float32),
        pltpu.VMEM_SHARED((NPAD, HD), jnp.float32),
    ] + [pltpu.SemaphoreType.DMA] * NBUF,
)

BN = 1000


def _mlp_core(xa, xb, aa, ab, w1_ref, b1_ref, w2_ref, b2_ref):
  h = jnp.concatenate([xa + aa, xb + ab], axis=1)
  h1 = jnp.dot(h, w1_ref[...], preferred_element_type=jnp.float32) + b1_ref[...]
  h1 = jnp.maximum(h1, 0.0)
  return jnp.dot(h1, w2_ref[...], preferred_element_type=jnp.float32) + b2_ref[...]


def _mlp_body(xa_ref, xb_ref, aa_ref, ab_ref, w1_ref, b1_ref, w2_ref, b2_ref,
              oa_ref, ob_ref):
  o = _mlp_core(xa_ref[...], xb_ref[...], aa_ref[0], ab_ref[0],
                w1_ref, b1_ref, w2_ref, b2_ref)
  o = jnp.maximum(o, 0.0)
  oa_ref[...] = o[:, :HD]
  ob_ref[...] = o[:, HD:]


def _row_specs():
  half = pl.BlockSpec((BN, HD), lambda i: (i, 0))
  agg0 = pl.BlockSpec((1, BN, HD), lambda i: (0, i, 0))
  agg1 = pl.BlockSpec((1, BN, HD), lambda i: (1, i, 0))

  def full(shape):
    return pl.BlockSpec(shape, lambda i: (0, 0))

  return half, agg0, agg1, full


def _make_mlp(first_layer):
  half, agg0, agg1, full = _row_specs()
  if first_layer:
    xa = pl.BlockSpec((BN, HD), lambda i: (i, 0))
    xb = pl.BlockSpec((BN, HD), lambda i: (i, 1))
  else:
    xa, xb = half, half
  return pl.pallas_call(
      _mlp_body,
      grid=(N // BN,),
      in_specs=[xa, xb, agg0, agg1,
                full((D, D)), full((1, D)), full((D, D)), full((1, D))],
      out_specs=[half, half],
      out_shape=[jax.ShapeDtypeStruct((N, HD), jnp.float32)] * 2,
  )


_mlp_first = _make_mlp(True)
_mlp_mid = _make_mlp(False)


def _mlp_pool_body(b_ref, xa_ref, xb_ref, aa_ref, ab_ref,
                   w1_ref, b1_ref, w2_ref, b2_ref, out_ref, acc_ref):
  i = pl.program_id(0)

  @pl.when(i == 0)
  def _():
    acc_ref[...] = jnp.zeros_like(acc_ref)

  o = _mlp_core(xa_ref[...], xb_ref[...], aa_ref[0], ab_ref[0],
                w1_ref, b1_ref, w2_ref, b2_ref)
  oh = (b_ref[...] == lax.broadcasted_iota(jnp.int32, (BN, G), 1)
        ).astype(jnp.float32)
  hx = jnp.concatenate([o, jnp.ones((BN, 128), jnp.float32)], axis=1)
  acc_ref[...] += lax.dot_general(oh, hx, (((0,), (0,)), ((), ())),
                                  preferred_element_type=jnp.float32)

  @pl.when(i == pl.num_programs(0) - 1)
  def _():
    out_ref[...] = acc_ref[:, :D] / jnp.maximum(acc_ref[:, D:D + 1], 1.0)


def _make_mlp_pool():
  half, agg0, agg1, full = _row_specs()
  return pl.pallas_call(
      _mlp_pool_body,
      grid=(N // BN,),
      in_specs=[pl.BlockSpec((BN, 1), lambda i: (i, 0)),
                half, half, agg0, agg1,
                full((D, D)), full((1, D)), full((D, D)), full((1, D))],
      out_specs=pl.BlockSpec((G, D), lambda i: (0, 0)),
      out_shape=jax.ShapeDtypeStruct((G, D), jnp.float32),
      scratch_shapes=[pltpu.VMEM((G, D + 128), jnp.float32)],
  )


_mlp_pool = _make_mlp_pool()


def kernel(x, edge_index, batch, W1_0, b1_0, W2_0, b2_0,
           W1_1, b1_1, W2_1, b2_1, W1_2, b1_2, W2_2, b2_2):
  src = edge_index[0].astype(jnp.int32)
  dst = edge_index[1].astype(jnp.int32)
  srcp = jnp.concatenate([src, jnp.zeros((EPAD - E,), jnp.int32)])
  dstp = jnp.concatenate([dst, jnp.full((EPAD - E,), N, jnp.int32)])
  src3 = srcp.reshape(NCHUNK, CH)
  dst3 = dstp.reshape(NCHUNK, CH)
  zeros = jnp.zeros((NPAD, HD), jnp.float32)
  batch2 = batch[:, None].astype(jnp.int32)

  xa, xb = x[:, :HD], x[:, HD:]
  agg = _sc_agg(xa, xb, src3, dst3, zeros)
  xa, xb = _mlp_first(x, x, agg, agg,
                      W1_0, b1_0.reshape(1, D), W2_0, b2_0.reshape(1, D))
  agg = _sc_agg(xa, xb, src3, dst3, zeros)
  xa, xb = _mlp_mid(xa, xb, agg, agg,
                    W1_1, b1_1.reshape(1, D), W2_1, b2_1.reshape(1, D))
  agg = _sc_agg(xa, xb, src3, dst3, zeros)
  return _mlp_pool(batch2, xa, xb, agg, agg,
                   W1_2, b1_2.reshape(1, D), W2_2, b2_2.reshape(1, D))


# speedup vs baseline: 1.1288x; 1.0915x over previous
import functools

import jax
import jax.numpy as jnp
from jax import lax
from jax.experimental import pallas as pl
from jax.experimental.pallas import tpu as pltpu
from jax.experimental.pallas import tpu_sc as plsc

N = 10000
E = 160000
D = 256
G = 64
HD = D // 2
NC, NS = 2, 16
CH = 128
NCHUNK = 1280
EPAD = NCHUNK * CH
RPT = NCHUNK // NS
NPAD = 10112
ZPT = NPAD // NS
NBUF = 2
PHASES = 2
PRPT = RPT // PHASES
NG = PRPT // NBUF


def _sc_agg_body(x2_hbm, src3_hbm, dst3_hbm, zeros_hbm, out_hbm,
                 idx_s, idx_d, rows, acc, *sems):
  c = lax.axis_index("c")
  s = lax.axis_index("s")
  pltpu.sync_copy(zeros_hbm.at[pl.ds(s * ZPT, ZPT)], acc.at[pl.ds(s * ZPT, ZPT)])
  base = s * RPT
  plsc.subcore_barrier()

  def gather(j, b):
    pltpu.async_copy(x2_hbm.at[idx_s.at[j]], rows.at[b], sems[b])

  for ph in range(PHASES):
    hbase = base + ph * PRPT
    pltpu.sync_copy(src3_hbm.at[c, pl.ds(hbase, PRPT)], idx_s)
    pltpu.sync_copy(dst3_hbm.at[pl.ds(hbase, PRPT)], idx_d)
    for b in range(NBUF):
      gather(b, b)

    def grp(g, carry):
      for b in range(NBUF):
        j = g * NBUF + b
        pltpu.make_async_copy(x2_hbm.at[idx_s.at[j]], rows.at[b],
                              sems[b]).wait()
        pltpu.sync_copy(rows.at[b], acc.at[idx_d.at[j]], add=True)

        @pl.when(g < NG - 1)
        def _():
          gather(j + NBUF, b)
      return carry

    lax.fori_loop(0, NG, grp, 0)

  plsc.subcore_barrier()
  pltpu.sync_copy(acc.at[pl.ds(s * ZPT, ZPT)], out_hbm.at[c, pl.ds(s * ZPT, ZPT)])


_sc_agg = pl.kernel(
    _sc_agg_body,
    out_type=jax.ShapeDtypeStruct((NC, NPAD, HD), jnp.float32),
    mesh=plsc.VectorSubcoreMesh(core_axis_name="c", subcore_axis_name="s",
                                num_cores=NC, num_subcores=NS),
    scratch_types=[
        pltpu.VMEM((PRPT, CH), jnp.int32),
        pltpu.VMEM((PRPT, CH), jnp.int32),
        pltpu.VMEM((NBUF, CH, HD), jnp.float32),
        pltpu.VMEM_SHARED((NPAD, HD), jnp.float32),
    ] + [pltpu.SemaphoreType.DMA] * NBUF,
)

BN = 1000


def _mlp_core(xa, xb, aa, ab, w1_ref, b1_ref, w2_ref, b2_ref):
  h = jnp.concatenate([xa + aa, xb + ab], axis=1)
  h1 = jnp.dot(h, w1_ref[...], preferred_element_type=jnp.float32) + b1_ref[...]
  h1 = jnp.maximum(h1, 0.0)
  return jnp.dot(h1, w2_ref[...], preferred_element_type=jnp.float32) + b2_ref[...]


def _mlp_body(xa_ref, xb_ref, aa_ref, ab_ref, w1_ref, b1_ref, w2_ref, b2_ref,
              oa_ref, ob_ref):
  o = _mlp_core(xa_ref[...], xb_ref[...], aa_ref[0], ab_ref[0],
                w1_ref, b1_ref, w2_ref, b2_ref)
  o = jnp.maximum(o, 0.0)
  oa_ref[...] = o[:, :HD]
  ob_ref[...] = o[:, HD:]


def _row_specs():
  half = pl.BlockSpec((BN, HD), lambda i: (i, 0))
  agg0 = pl.BlockSpec((1, BN, HD), lambda i: (0, i, 0))
  agg1 = pl.BlockSpec((1, BN, HD), lambda i: (1, i, 0))

  def full(shape):
    return pl.BlockSpec(shape, lambda i: (0, 0))

  return half, agg0, agg1, full


def _make_mlp(first_layer):
  half, agg0, agg1, full = _row_specs()
  if first_layer:
    xa = pl.BlockSpec((BN, HD), lambda i: (i, 0))
    xb = pl.BlockSpec((BN, HD), lambda i: (i, 1))
  else:
    xa, xb = half, half
  return pl.pallas_call(
      _mlp_body,
      grid=(N // BN,),
      in_specs=[xa, xb, agg0, agg1,
                full((D, D)), full((1, D)), full((D, D)), full((1, D))],
      out_specs=[half, half],
      out_shape=[jax.ShapeDtypeStruct((N, HD), jnp.float32)] * 2,
  )


_mlp_first = _make_mlp(True)
_mlp_mid = _make_mlp(False)


def _mlp_pool_body(b_ref, xa_ref, xb_ref, aa_ref, ab_ref,
                   w1_ref, b1_ref, w2_ref, b2_ref, out_ref, acc_ref):
  i = pl.program_id(0)

  @pl.when(i == 0)
  def _():
    acc_ref[...] = jnp.zeros_like(acc_ref)

  o = _mlp_core(xa_ref[...], xb_ref[...], aa_ref[0], ab_ref[0],
                w1_ref, b1_ref, w2_ref, b2_ref)
  oh = (b_ref[...] == lax.broadcasted_iota(jnp.int32, (BN, G), 1)
        ).astype(jnp.float32)
  hx = jnp.concatenate([o, jnp.ones((BN, 128), jnp.float32)], axis=1)
  acc_ref[...] += lax.dot_general(oh, hx, (((0,), (0,)), ((), ())),
                                  preferred_element_type=jnp.float32)

  @pl.when(i == pl.num_programs(0) - 1)
  def _():
    out_ref[...] = acc_ref[:, :D] / jnp.maximum(acc_ref[:, D:D + 1], 1.0)


def _make_mlp_pool():
  half, agg0, agg1, full = _row_specs()
  return pl.pallas_call(
      _mlp_pool_body,
      grid=(N // BN,),
      in_specs=[pl.BlockSpec((BN, 1), lambda i: (i, 0)),
                half, half, agg0, agg1,
                full((D, D)), full((1, D)), full((D, D)), full((1, D))],
      out_specs=pl.BlockSpec((G, D), lambda i: (0, 0)),
      out_shape=jax.ShapeDtypeStruct((G, D), jnp.float32),
      scratch_shapes=[pltpu.VMEM((G, D + 128), jnp.float32)],
  )


_mlp_pool = _make_mlp_pool()


def kernel(x, edge_index, batch, W1_0, b1_0, W2_0, b2_0,
           W1_1, b1_1, W2_1, b2_1, W1_2, b1_2, W2_2, b2_2):
  src = edge_index[0].astype(jnp.int32)
  dst = edge_index[1].astype(jnp.int32)
  srcp = jnp.concatenate([src, jnp.zeros((EPAD - E,), jnp.int32)])
  dstp = jnp.concatenate([dst, jnp.full((EPAD - E,), N, jnp.int32)])
  src3 = jnp.stack([srcp, srcp + N]).reshape(NC, NCHUNK, CH)
  dst3 = dstp.reshape(NCHUNK, CH)
  zeros = jnp.zeros((NPAD, HD), jnp.float32)
  batch2 = batch[:, None].astype(jnp.int32)

  xa, xb = x[:, :HD], x[:, HD:]
  agg = _sc_agg(jnp.concatenate([xa, xb], axis=0), src3, dst3, zeros)
  xa, xb = _mlp_first(x, x, agg, agg,
                      W1_0, b1_0.reshape(1, D), W2_0, b2_0.reshape(1, D))
  agg = _sc_agg(jnp.concatenate([xa, xb], axis=0), src3, dst3, zeros)
  xa, xb = _mlp_mid(xa, xb, agg, agg,
                    W1_1, b1_1.reshape(1, D), W2_1, b2_1.reshape(1, D))
  agg = _sc_agg(jnp.concatenate([xa, xb], axis=0), src3, dst3, zeros)
  return _mlp_pool(batch2, xa, xb, agg, agg,
                   W1_2, b1_2.reshape(1, D), W2_2, b2_2.reshape(1, D))
